# baseline (speedup 1.0000x reference)
import functools

import jax
import jax.numpy as jnp
from jax import lax
from jax.experimental import pallas as pl
from jax.experimental.pallas import tpu as pltpu
from jax.experimental.pallas import tpu_sc as plsc

BATCH = 16384
EMBED_DIM = 64
PAIR = 2 * EMBED_DIM
NROWS = 100000
NPANEL = 782
H1 = 128
H2 = 64


@functools.cache
def _sc_info():
    info = plsc.get_sparse_core_info()
    return info.num_cores, info.num_subcores


def _mesh():
    return plsc.VectorSubcoreMesh(core_axis_name="c", subcore_axis_name="s")


@functools.cache
def _build_retile():
    nc, ns = _sc_info()
    nw = nc * ns

    @functools.partial(
        pl.kernel,
        mesh=_mesh(),
        compiler_params=pltpu.CompilerParams(needs_layout_passes=False),
        out_type=jax.ShapeDtypeStruct((NPANEL * 8192,), jnp.float32),
        scratch_types=[
            pltpu.VMEM((EMBED_DIM, 128), jnp.float32),
            pltpu.VMEM((8192,), jnp.float32),
            pltpu.SemaphoreType.DMA,
        ],
    )
    def retile(tab_t, out1d, panel_v, stage_v, sem):
        wid = lax.axis_index("s") * nc + lax.axis_index("c")
        lo = (wid * NPANEL) // nw
        hi = ((wid + 1) * NPANEL) // nw
        d16 = lax.iota(jnp.int32, 16)

        def panel_body(p, _):
            c0 = pl.multiple_of(p * 128, 128)
            pltpu.async_copy(
                tab_t.at[:, pl.ds(c0, 128)], panel_v, sem).wait()

            def col_body(c, _):
                csplat = jnp.full((16,), c, jnp.int32)
                loff = (c >> 1) * 128 + (c & 1) * EMBED_DIM
                for k in range(4):
                    v = plsc.load_gather(panel_v, [d16 + 16 * k, csplat])
                    off = pl.multiple_of(loff + 16 * k, 8)
                    stage_v[pl.ds(off, 16)] = v
                return 0

            lax.fori_loop(0, 128, col_body, 0)
            o0 = pl.multiple_of(p * 8192, 8)
            pltpu.sync_copy(stage_v, out1d.at[pl.ds(o0, 8192)])
            return 0

        lax.fori_loop(lo, hi, panel_body, 0)

    return retile


@functools.cache
def _build_gather_pairs():
    nc, ns = _sc_info()
    bpw = BATCH // (nc * ns)

    @functools.partial(
        pl.kernel,
        mesh=_mesh(),
        out_type=jax.ShapeDtypeStruct((BATCH * PAIR,), jnp.float32),
        scratch_types=[
            pltpu.VMEM((bpw,), jnp.int32),
            pltpu.VMEM((bpw * PAIR,), jnp.float32),
            pltpu.SemaphoreType.DMA,
        ],
    )
    def gather_pairs(pairs1d, idx, out1d, idx_v, stage_v, sem):
        wid = lax.axis_index("s") * nc + lax.axis_index("c")
        base = wid * bpw
        pltpu.sync_copy(idx.at[pl.ds(base, bpw)], idx_v)

        def body(b, _):
            k = b * 16
            roff = (idx_v[pl.ds(k, 16)] >> 1) * PAIR
            for j in range(16):
                src = pl.multiple_of(roff[j], 8)
                pltpu.async_copy(
                    pairs1d.at[pl.ds(src, PAIR)],
                    stage_v.at[pl.ds((k + j) * PAIR, PAIR)], sem)
            return 0

        lax.fori_loop(0, bpw // 16, body, 0)
        pltpu.make_async_copy(pairs1d.at[pl.ds(0, bpw * PAIR)], stage_v,
                              sem).wait()
        pltpu.sync_copy(stage_v, out1d.at[pl.ds(base * PAIR, bpw * PAIR)])

    return gather_pairs


@functools.cache
def _build_gather_tiled():
    nc, ns = _sc_info()
    bpw = BATCH // (nc * ns)

    @functools.partial(
        pl.kernel,
        mesh=_mesh(),
        out_type=jax.ShapeDtypeStruct((BATCH, EMBED_DIM), jnp.float32),
        scratch_types=[
            pltpu.VMEM((bpw,), jnp.int32),
            pltpu.VMEM((bpw, EMBED_DIM), jnp.float32),
            pltpu.SemaphoreType.DMA,
        ],
    )
    def gather_rows(tab, idx, out, idx_v, rows_v, sem):
        wid = lax.axis_index("s") * nc + lax.axis_index("c")
        base = wid * bpw
        pltpu.sync_copy(idx.at[pl.ds(base, bpw)], idx_v)

        def body(b, _):
            k = b * 16
            v = idx_v[pl.ds(k, 16)]
            for j in range(16):
                pltpu.async_copy(
                    tab.at[pl.ds(v[j], 1)], rows_v.at[pl.ds(k + j, 1)], sem)
            return 0

        lax.fori_loop(0, bpw // 16, body, 0)
        pltpu.make_async_copy(tab.at[pl.ds(0, bpw)], rows_v, sem).wait()
        pltpu.sync_copy(rows_v, out.at[pl.ds(base, bpw)])

    return gather_rows


def _mlp_body(up_ref, se_ref, upar_ref, w1u_ref, w1s_ref, b1_ref, w2_ref,
              b2_ref, w3_ref, b3_ref, out_ref):
    dn1 = (((1,), (1,)), ((), ()))
    dn0 = (((1,), (0,)), ((), ()))
    up = up_ref[...]
    ue = jnp.where(upar_ref[...] > 0.5, up[:, EMBED_DIM:], up[:, :EMBED_DIM])
    h = lax.dot_general(w1u_ref[...], ue, dn1,
                        preferred_element_type=jnp.float32)
    h += lax.dot_general(w1s_ref[...], se_ref[...], dn1,
                         preferred_element_type=jnp.float32)
    h = jnp.maximum(h + b1_ref[...], 0.0)
    h = lax.dot_general(w2_ref[...], h, dn0,
                        preferred_element_type=jnp.float32)
    h = jnp.maximum(h + b2_ref[...], 0.0)
    o = lax.dot_general(w3_ref[...], h, dn0,
                        preferred_element_type=jnp.float32)
    o = o + b3_ref[...]
    out_ref[...] = 1.0 / (1.0 + jnp.exp(-o))


def _mlp(up, se, upar, W1u, W1s, b1, W2, b2, W3, b3):
    blk = 4096
    grid = BATCH // blk
    full = lambda shape: pl.BlockSpec(shape, lambda i: (0, 0))
    return pl.pallas_call(
        _mlp_body,
        grid=(grid,),
        in_specs=[
            pl.BlockSpec((blk, PAIR), lambda i: (i, 0)),
            pl.BlockSpec((blk, EMBED_DIM), lambda i: (i, 0)),
            pl.BlockSpec((blk, 1), lambda i: (i, 0)),
            full((H1, EMBED_DIM)),
            full((H1, EMBED_DIM)),
            full((H1, 1)),
            full((H2, H1)),
            full((H2, 1)),
            full((1, H2)),
            full((1, 1)),
        ],
        out_specs=pl.BlockSpec((1, blk), lambda i: (0, i)),
        out_shape=jax.ShapeDtypeStruct((1, BATCH), jnp.float32),
    )(up, se, upar, W1u, W1s, b1, W2, b2, W3, b3)


def kernel(user, song, user_table, song_table, W1, b1, W2, b2, W3, b3):
    user = user.astype(jnp.int32)
    song = song.astype(jnp.int32)
    se = _build_gather_tiled()(song_table, song)
    pairs = _build_retile()(user_table.T)
    u1d = _build_gather_pairs()(pairs, user)
    up = u1d.reshape(BATCH, PAIR)
    upar = (user & 1).astype(jnp.float32).reshape(BATCH, 1)
    out = _mlp(up, se, upar, W1[:, :EMBED_DIM], W1[:, EMBED_DIM:],
               b1.reshape(H1, 1), W2, b2.reshape(H2, 1), W3, b3.reshape(1, 1))
    return out.reshape(BATCH, 1)


# speedup vs baseline: 2.2771x; 2.2771x over previous
import functools

import jax
import jax.numpy as jnp
from jax import lax
from jax.experimental import pallas as pl
from jax.experimental.pallas import tpu as pltpu
from jax.experimental.pallas import tpu_sc as plsc

BATCH = 16384
EMBED_DIM = 64
H1 = 128
H2 = 64


@functools.cache
def _sc_info():
    info = plsc.get_sparse_core_info()
    return info.num_cores, info.num_subcores


@functools.cache
def _build_gather_tiled():
    nc, ns = _sc_info()
    bpw = BATCH // (nc * ns)
    mesh = plsc.VectorSubcoreMesh(core_axis_name="c", subcore_axis_name="s")

    @functools.partial(
        pl.kernel,
        mesh=mesh,
        out_type=jax.ShapeDtypeStruct((BATCH, EMBED_DIM), jnp.float32),
        scratch_types=[
            pltpu.VMEM((bpw,), jnp.int32),
            pltpu.VMEM((bpw, EMBED_DIM), jnp.float32),
            pltpu.SemaphoreType.DMA,
        ],
    )
    def gather_rows(tab, idx, out, idx_v, rows_v, sem):
        wid = lax.axis_index("s") * nc + lax.axis_index("c")
        base = wid * bpw
        pltpu.sync_copy(idx.at[pl.ds(base, bpw)], idx_v)

        def body(b, _):
            k = b * 16
            v = idx_v[pl.ds(k, 16)]
            for j in range(16):
                pltpu.async_copy(
                    tab.at[pl.ds(v[j], 1)], rows_v.at[pl.ds(k + j, 1)], sem)
            return 0

        lax.fori_loop(0, bpw // 16, body, 0)
        pltpu.make_async_copy(tab.at[pl.ds(0, bpw)], rows_v, sem).wait()
        pltpu.sync_copy(rows_v, out.at[pl.ds(base, bpw)])

    return gather_rows


def _mlp_body(ue_ref, se_ref, w1u_ref, w1s_ref, b1_ref, w2_ref, b2_ref,
              w3_ref, b3_ref, out_ref):
    dn1 = (((1,), (1,)), ((), ()))
    dn0 = (((1,), (0,)), ((), ()))
    h = lax.dot_general(w1u_ref[...], ue_ref[...], dn1,
                        preferred_element_type=jnp.float32)
    h += lax.dot_general(w1s_ref[...], se_ref[...], dn1,
                         preferred_element_type=jnp.float32)
    h = jnp.maximum(h + b1_ref[...], 0.0)
    h = lax.dot_general(w2_ref[...], h, dn0,
                        preferred_element_type=jnp.float32)
    h = jnp.maximum(h + b2_ref[...], 0.0)
    o = lax.dot_general(w3_ref[...], h, dn0,
                        preferred_element_type=jnp.float32)
    o = o + b3_ref[...]
    out_ref[...] = 1.0 / (1.0 + jnp.exp(-o))


def _mlp(ue, se, W1u, W1s, b1, W2, b2, W3, b3):
    blk = 4096
    grid = BATCH // blk
    full = lambda shape: pl.BlockSpec(shape, lambda i: (0, 0))
    return pl.pallas_call(
        _mlp_body,
        grid=(grid,),
        in_specs=[
            pl.BlockSpec((blk, EMBED_DIM), lambda i: (i, 0)),
            pl.BlockSpec((blk, EMBED_DIM), lambda i: (i, 0)),
            full((H1, EMBED_DIM)),
            full((H1, EMBED_DIM)),
            full((H1, 1)),
            full((H2, H1)),
            full((H2, 1)),
            full((1, H2)),
            full((1, 1)),
        ],
        out_specs=pl.BlockSpec((1, blk), lambda i: (0, i)),
        out_shape=jax.ShapeDtypeStruct((1, BATCH), jnp.float32),
    )(ue, se, W1u, W1s, b1, W2, b2, W3, b3)


def kernel(user, song, user_table, song_table, W1, b1, W2, b2, W3, b3):
    ue = _build_gather_tiled()(user_table, user.astype(jnp.int32))
    se = _build_gather_tiled()(song_table, song.astype(jnp.int32))
    out = _mlp(ue, se, W1[:, :EMBED_DIM], W1[:, EMBED_DIM:], b1.reshape(H1, 1),
               W2, b2.reshape(H2, 1), W3, b3.reshape(1, 1))
    return out.reshape(BATCH, 1)
